# baseline (speedup 1.0000x reference)
import functools

import jax
import jax.numpy as jnp
from jax import lax
from jax.experimental import pallas as pl
from jax.experimental.pallas import tpu as pltpu
from jax.experimental.pallas import tpu_sc as plsc

N_ENSEMBLE = 10
VOCAB = 100000
DIM = 64
B, L = 4096, 50
N = B * L

_info = plsc.get_sparse_core_info()
NC, NS = _info.num_cores, _info.num_subcores
NW = NC * NS
PER_W = N // NW
CH = 128
NCH = PER_W // CH

_mesh = plsc.VectorSubcoreMesh(core_axis_name="c", subcore_axis_name="s")


@functools.partial(
    pl.kernel,
    mesh=_mesh,
    compiler_params=pltpu.CompilerParams(use_tc_tiling_on_sc=False),
    out_type=jax.ShapeDtypeStruct((N, DIM), jnp.float32),
    scratch_types=[
        pltpu.VMEM((NCH, CH), jnp.int32),
        pltpu.VMEM((CH, DIM), jnp.float32),
        pltpu.SemaphoreType.DMA,
    ],
)
def _table_lookup(idx_hbm, tab_hbm, out, idx_v, rows_v, sem):
    wid = lax.axis_index("s") * NC + lax.axis_index("c")
    base = wid * PER_W
    pltpu.sync_copy(idx_hbm.at[wid], idx_v)

    def chunk_body(c, _):
        pltpu.async_copy(tab_hbm.at[idx_v.at[c]], rows_v, sem).wait()
        pltpu.sync_copy(rows_v, out.at[pl.ds(base + c * CH, CH)])
        return 0

    lax.fori_loop(0, NCH, chunk_body, 0)


def kernel(indices, tables):
    idx = indices.astype(jnp.int32).reshape(NW, NCH, CH)
    outs = []
    for t in range(N_ENSEMBLE):
        raw = _table_lookup(idx, tables[t])
        outs.append(raw.reshape(B, L, DIM) * 8.0)
    return tuple(outs)


# speedup vs baseline: 1.1560x; 1.1560x over previous
import functools

import jax
import jax.numpy as jnp
from jax import lax
from jax.experimental import pallas as pl
from jax.experimental.pallas import tpu as pltpu
from jax.experimental.pallas import tpu_sc as plsc

N_ENSEMBLE = 10
VOCAB = 100000
DIM = 64
B, L = 4096, 50

_info = plsc.get_sparse_core_info()
NC, NS = _info.num_cores, _info.num_subcores
NW = NC * NS
BW = B // NW

_mesh = plsc.VectorSubcoreMesh(core_axis_name="c", subcore_axis_name="s")


@functools.partial(
    pl.kernel,
    mesh=_mesh,
    compiler_params=pltpu.CompilerParams(needs_layout_passes=False),
    out_type=jax.ShapeDtypeStruct((L, DIM // 8, NW, 8, BW), jnp.float32),
    scratch_types=(
        [pltpu.VMEM((L, BW), jnp.int32)]
        + [pltpu.VMEM((BW, 2 * DIM), jnp.float32) for _ in range(2)]
        + [pltpu.VMEM((DIM // 8, 8, BW), jnp.float32) for _ in range(2)]
        + [pltpu.SemaphoreType.DMA for _ in range(4)]
    ),
)
def _table_lookup(idx_hbm, tab_hbm, out, idx_v, rows0, rows1, tr0, tr1,
                  g0, g1, w0, w1):
    rows = (rows0, rows1)
    trs = (tr0, tr1)
    gsem = (g0, g1)
    wsem = (w0, w1)
    wid = lax.axis_index("s") * NC + lax.axis_index("c")
    pltpu.sync_copy(idx_hbm.at[:, wid, :], idx_v)

    iota = lax.iota(jnp.int32, 16)
    row_ids = [g * 16 + iota for g in range(8)]

    def start_g(l, p):
        pltpu.async_copy(tab_hbm.at[idx_v.at[l]], rows[p], gsem[p])

    def wait_g(p):
        pltpu.make_async_copy(tab_hbm.at[idx_v.at[0]], rows[p], gsem[p]).wait()

    def start_s(l, p):
        pltpu.async_copy(trs[p], out.at[l, :, wid, :, :], wsem[p])

    def wait_s(p):
        pltpu.make_async_copy(trs[p], out.at[0, :, wid, :, :], wsem[p]).wait()

    def transpose(p):
        def dbody(d, _):
            col = jnp.zeros((16,), jnp.int32) + d
            for g in range(8):
                v = plsc.load_gather(rows[p], [row_ids[g], col])
                trs[p][d // 8, d % 8, pl.ds(g * 16, 16)] = v
            return 0

        lax.fori_loop(0, DIM, dbody, 0)

    start_g(0, 0)
    for l in (0, 1):
        wait_g(l % 2)
        start_g(l + 1, (l + 1) % 2)
        transpose(l % 2)
        start_s(l, l % 2)

    def pair_body(m, _):
        for k in (0, 1):
            l = 2 * m + k
            wait_g(k)
            start_g(l + 1, 1 - k)
            wait_s(k)
            transpose(k)
            start_s(l, k)
        return 0

    lax.fori_loop(1, (L - 2) // 2, pair_body, 0)

    for l in (L - 2, L - 1):
        p = l % 2
        wait_g(p)
        if l + 1 < L:
            start_g(l + 1, 1 - p)
        wait_s(p)
        transpose(p)
        start_s(l, p)
    wait_s(0)
    wait_s(1)


def kernel(indices, tables):
    idx3 = jnp.transpose(indices.astype(jnp.int32)).reshape(L, NW, BW)
    outs = []
    for t in range(N_ENSEMBLE):
        tab_wide = jnp.pad(tables[t] * 8.0, ((0, 0), (0, 128 - DIM)))
        raw = _table_lookup(idx3, tab_wide)
        out = raw.transpose(2, 4, 0, 1, 3).reshape(B, L, DIM)
        outs.append(out)
    return tuple(outs)
